# baseline (speedup 1.0000x reference)
import functools

import jax
import jax.numpy as jnp
from jax import lax
from jax.experimental import pallas as pl
from jax.experimental.pallas import tpu as pltpu
from jax.experimental.pallas import tpu_sc as plsc

D = 50
DP = 64
H = 64


def _make_sc_gather(n_rows, d, n_workers=32, k=8):
    sg_rows = k * 128
    n_sg = n_rows // sg_rows
    assert n_sg * sg_rows == n_rows
    n_outer = (n_sg + n_workers - 1) // n_workers

    mesh = plsc.VectorSubcoreMesh(core_axis_name="c", subcore_axis_name="s")

    @functools.partial(
        pl.kernel,
        mesh=mesh,
        compiler_params=pltpu.CompilerParams(use_tc_tiling_on_sc=False),
        out_type=jax.ShapeDtypeStruct((n_rows, d), jnp.float32),
        scratch_types=[
            pltpu.VMEM((k, 128), jnp.int32),
            pltpu.VMEM((sg_rows, d), jnp.float32),
            pltpu.SemaphoreType.DMA,
        ],
    )
    def gather(emb_hbm, idx_hbm, out_hbm, idx_v, rows_v, sem):
        wid = lax.axis_index("s") * 2 + lax.axis_index("c")

        def body(j, carry):
            sg = j * n_workers + wid

            @pl.when(sg < n_sg)
            def _():
                ir0 = pl.multiple_of(sg * k, 8)
                pltpu.sync_copy(idx_hbm.at[pl.ds(ir0, k)], idx_v)
                cps = [
                    pltpu.async_copy(
                        emb_hbm.at[idx_v.at[j2]],
                        rows_v.at[pl.ds(j2 * 128, 128)],
                        sem,
                    )
                    for j2 in range(k)
                ]
                for cp in cps:
                    cp.wait()
                r0 = pl.multiple_of(sg * sg_rows, 8)
                pltpu.sync_copy(rows_v, out_hbm.at[pl.ds(r0, sg_rows)])

            return carry

        lax.fori_loop(0, n_outer, body, 0)

    return gather


def _make_lstm_call(b2, t_span, tc, first, last, interpret=False):
    nt = t_span // tc
    half = b2 // 2

    def body(*refs):
        if first:
            (e_ref, wt_ref, ut_ref, bt_ref, wdt_ref, bdt_ref), rest = refs[:6], refs[6:]
        else:
            (e_ref, wt_ref, ut_ref, bt_ref, wdt_ref, bdt_ref, h0_ref, c0_ref), rest = refs[:8], refs[8:]
        if last:
            (out_ref, h_ref, c_ref, xw_ref) = rest
        else:
            (ho_ref, co_ref, h_ref, c_ref, xw_ref) = rest
        t_idx = pl.program_id(0)

        @pl.when(t_idx == 0)
        def _():
            if first:
                h_ref[...] = jnp.zeros((H, b2), jnp.float32)
                c_ref[...] = jnp.zeros((H, b2), jnp.float32)
            else:
                h_ref[...] = h0_ref[...]
                c_ref[...] = c0_ref[...]

        wt = wt_ref[...]
        ut = ut_ref[...]
        bt = bt_ref[...]

        xall = e_ref[...].reshape(tc * b2, DP)
        xw_ref[...] = lax.dot_general(wt, xall, (((1,), (1,)), ((), ())),
                                      preferred_element_type=jnp.float32)

        def step(tt, hc):
            h, c = hc
            z = (xw_ref[:, pl.ds(tt * b2, b2)]
                 + jnp.dot(ut, h, preferred_element_type=jnp.float32)
                 + bt)
            gi = 0.5 + 0.5 * jnp.tanh(0.5 * z[0 * H:1 * H])
            gf = 0.5 + 0.5 * jnp.tanh(0.5 * z[1 * H:2 * H])
            gg = jnp.tanh(z[2 * H:3 * H])
            go = 0.5 + 0.5 * jnp.tanh(0.5 * z[3 * H:4 * H])
            c = gf * c + gi * gg
            h = go * jnp.tanh(c)
            return (h, c)

        h, c = lax.fori_loop(0, tc, step, (h_ref[...], c_ref[...]))
        h_ref[...] = h
        c_ref[...] = c

        @pl.when(t_idx == nt - 1)
        def _():
            if last:
                merged = jnp.concatenate([h[:, :half], h[:, half:]], axis=0)
                logits = (jnp.dot(wdt_ref[...], merged,
                                  preferred_element_type=jnp.float32)
                          + bdt_ref[...])
                m = jnp.max(logits, axis=0, keepdims=True)
                p = jnp.exp(logits - m)
                out_ref[...] = p / jnp.sum(p, axis=0, keepdims=True)
            else:
                ho_ref[...] = h
                co_ref[...] = c

    in_specs = [
        pl.BlockSpec((tc, b2, DP), lambda t: (t, 0, 0)),
        pl.BlockSpec((4 * H, DP), lambda t: (0, 0)),
        pl.BlockSpec((4 * H, H), lambda t: (0, 0)),
        pl.BlockSpec((4 * H, 1), lambda t: (0, 0)),
        pl.BlockSpec((8, 2 * H), lambda t: (0, 0)),
        pl.BlockSpec((8, 1), lambda t: (0, 0)),
    ]
    if not first:
        in_specs += [
            pl.BlockSpec((H, b2), lambda t: (0, 0)),
            pl.BlockSpec((H, b2), lambda t: (0, 0)),
        ]
    if last:
        out_specs = pl.BlockSpec((8, half), lambda t: (0, 0))
        out_shape = jax.ShapeDtypeStruct((8, half), jnp.float32)
    else:
        out_specs = [pl.BlockSpec((H, b2), lambda t: (0, 0))] * 2
        out_shape = [jax.ShapeDtypeStruct((H, b2), jnp.float32)] * 2

    return pl.pallas_call(
        body,
        grid=(nt,),
        in_specs=in_specs,
        out_specs=out_specs,
        out_shape=out_shape,
        scratch_shapes=[
            pltpu.VMEM((H, b2), jnp.float32),
            pltpu.VMEM((H, b2), jnp.float32),
            pltpu.VMEM((4 * H, tc * b2), jnp.float32),
        ],
        interpret=interpret,
    )


def _prep_weights(W, U, b, Wd, bd):
    Wt = jnp.pad(W, ((0, DP - W.shape[0]), (0, 0))).T
    Ut = U.T
    bt = b.reshape(-1, 1)
    Wdt = jnp.pad(Wd.T, ((0, 5), (0, 0)))
    bdt = jnp.concatenate([bd, jnp.full((5,), -1e30, bd.dtype)]).reshape(8, 1)
    return Wt, Ut, bt, Wdt, bdt


def _build_indices(input_1, input_2):
    return jnp.concatenate([input_1.T, input_2.T], axis=1).reshape(-1, 128).astype(jnp.int32)


def kernel(input_1, input_2, embed, W, U, b, Wd, bd):
    bsz, t_total = input_1.shape
    b2 = 2 * bsz
    n_rows = b2 * t_total

    emb_pad = jnp.pad(embed, ((0, 0), (0, DP - embed.shape[1])))
    Wt, Ut, bt, Wdt, bdt = _prep_weights(W, U, b, Wd, bd)
    idx = _build_indices(input_1, input_2)

    n_phases = 2
    t_span = t_total // n_phases
    rows_span = b2 * t_span
    idx_rows_span = rows_span // 128
    g = _make_sc_gather(rows_span, DP)
    es = [
        g(emb_pad, idx[p * idx_rows_span:(p + 1) * idx_rows_span])
        .reshape(t_span, b2, DP)
        for p in range(n_phases)
    ]

    h, c = _make_lstm_call(b2, t_span, 5, first=True, last=False)(
        es[0], Wt, Ut, bt, Wdt, bdt)
    for p in range(1, n_phases - 1):
        h, c = _make_lstm_call(b2, t_span, 5, first=False, last=False)(
            es[p], Wt, Ut, bt, Wdt, bdt, h, c)
    out_t = _make_lstm_call(b2, t_span, 5, first=False, last=True)(
        es[-1], Wt, Ut, bt, Wdt, bdt, h, c)
    return out_t[:3].T


# speedup vs baseline: 1.0350x; 1.0350x over previous
import functools

import jax
import jax.numpy as jnp
from jax import lax
from jax.experimental import pallas as pl
from jax.experimental.pallas import tpu as pltpu
from jax.experimental.pallas import tpu_sc as plsc

D = 50
DP = 64
H = 64


def _make_sc_gather(n_rows, d, n_workers=32, k=8):
    sg_rows = k * 128
    n_sg = n_rows // sg_rows
    assert n_sg * sg_rows == n_rows
    n_outer = (n_sg + n_workers - 1) // n_workers

    mesh = plsc.VectorSubcoreMesh(core_axis_name="c", subcore_axis_name="s")

    @functools.partial(
        pl.kernel,
        mesh=mesh,
        compiler_params=pltpu.CompilerParams(use_tc_tiling_on_sc=False),
        out_type=jax.ShapeDtypeStruct((n_rows, d), jnp.float32),
        scratch_types=[
            pltpu.VMEM((k, 128), jnp.int32),
            pltpu.VMEM((sg_rows, d), jnp.float32),
            pltpu.SemaphoreType.DMA,
        ],
    )
    def gather(emb_hbm, idx_hbm, out_hbm, idx_v, rows_v, sem):
        wid = lax.axis_index("s") * 2 + lax.axis_index("c")

        def body(j, carry):
            sg = j * n_workers + wid

            @pl.when(sg < n_sg)
            def _():
                ir0 = pl.multiple_of(sg * k, 8)
                pltpu.sync_copy(idx_hbm.at[pl.ds(ir0, k)], idx_v)
                cps = [
                    pltpu.async_copy(
                        emb_hbm.at[idx_v.at[j2]],
                        rows_v.at[pl.ds(j2 * 128, 128)],
                        sem,
                    )
                    for j2 in range(k)
                ]
                for cp in cps:
                    cp.wait()
                r0 = pl.multiple_of(sg * sg_rows, 8)
                pltpu.sync_copy(rows_v, out_hbm.at[pl.ds(r0, sg_rows)])

            return carry

        lax.fori_loop(0, n_outer, body, 0)

    return gather


def _make_lstm_call(b2, t_span, tc, first, last, interpret=False):
    nt = t_span // tc
    half = b2 // 2

    def body(*refs):
        if first:
            (e_ref, wt_ref, ut_ref, bt_ref, wdt_ref, bdt_ref), rest = refs[:6], refs[6:]
        else:
            (e_ref, wt_ref, ut_ref, bt_ref, wdt_ref, bdt_ref, h0_ref, c0_ref), rest = refs[:8], refs[8:]
        if last:
            (out_ref, h_ref, c_ref) = rest
        else:
            (ho_ref, co_ref, h_ref, c_ref) = rest
        t_idx = pl.program_id(0)

        @pl.when(t_idx == 0)
        def _():
            if first:
                h_ref[...] = jnp.zeros((H, b2), jnp.float32)
                c_ref[...] = jnp.zeros((H, b2), jnp.float32)
            else:
                h_ref[...] = h0_ref[...]
                c_ref[...] = c0_ref[...]

        wt = wt_ref[...]
        ut = ut_ref[...]
        bt = bt_ref[...]

        def step(tt, hc):
            h, c = hc
            x = e_ref[tt]
            z = lax.dot_general(wt, x, (((1,), (1,)), ((), ())),
                                preferred_element_type=jnp.float32)
            z = z + jnp.dot(ut, h, preferred_element_type=jnp.float32) + bt
            gi = 0.5 + 0.5 * jnp.tanh(0.5 * z[0 * H:1 * H])
            gf = 0.5 + 0.5 * jnp.tanh(0.5 * z[1 * H:2 * H])
            gg = jnp.tanh(z[2 * H:3 * H])
            go = 0.5 + 0.5 * jnp.tanh(0.5 * z[3 * H:4 * H])
            c = gf * c + gi * gg
            h = go * jnp.tanh(c)
            return (h, c)

        h, c = lax.fori_loop(0, tc, step, (h_ref[...], c_ref[...]))
        h_ref[...] = h
        c_ref[...] = c

        @pl.when(t_idx == nt - 1)
        def _():
            if last:
                merged = jnp.concatenate([h[:, :half], h[:, half:]], axis=0)
                logits = (jnp.dot(wdt_ref[...], merged,
                                  preferred_element_type=jnp.float32)
                          + bdt_ref[...])
                m = jnp.max(logits, axis=0, keepdims=True)
                p = jnp.exp(logits - m)
                out_ref[...] = p / jnp.sum(p, axis=0, keepdims=True)
            else:
                ho_ref[...] = h
                co_ref[...] = c

    in_specs = [
        pl.BlockSpec((tc, b2, DP), lambda t: (t, 0, 0)),
        pl.BlockSpec((4 * H, DP), lambda t: (0, 0)),
        pl.BlockSpec((4 * H, H), lambda t: (0, 0)),
        pl.BlockSpec((4 * H, 1), lambda t: (0, 0)),
        pl.BlockSpec((8, 2 * H), lambda t: (0, 0)),
        pl.BlockSpec((8, 1), lambda t: (0, 0)),
    ]
    if not first:
        in_specs += [
            pl.BlockSpec((H, b2), lambda t: (0, 0)),
            pl.BlockSpec((H, b2), lambda t: (0, 0)),
        ]
    if last:
        out_specs = pl.BlockSpec((8, half), lambda t: (0, 0))
        out_shape = jax.ShapeDtypeStruct((8, half), jnp.float32)
    else:
        out_specs = [pl.BlockSpec((H, b2), lambda t: (0, 0))] * 2
        out_shape = [jax.ShapeDtypeStruct((H, b2), jnp.float32)] * 2

    return pl.pallas_call(
        body,
        grid=(nt,),
        in_specs=in_specs,
        out_specs=out_specs,
        out_shape=out_shape,
        scratch_shapes=[
            pltpu.VMEM((H, b2), jnp.float32),
            pltpu.VMEM((H, b2), jnp.float32),
        ],
        interpret=interpret,
    )


def _prep_weights(W, U, b, Wd, bd):
    Wt = jnp.pad(W, ((0, DP - W.shape[0]), (0, 0))).T
    Ut = U.T
    bt = b.reshape(-1, 1)
    Wdt = jnp.pad(Wd.T, ((0, 5), (0, 0)))
    bdt = jnp.concatenate([bd, jnp.full((5,), -1e30, bd.dtype)]).reshape(8, 1)
    return Wt, Ut, bt, Wdt, bdt


def _build_indices(input_1, input_2):
    return jnp.concatenate([input_1.T, input_2.T], axis=1).reshape(-1, 128).astype(jnp.int32)


def kernel(input_1, input_2, embed, W, U, b, Wd, bd):
    bsz, t_total = input_1.shape
    b2 = 2 * bsz
    n_rows = b2 * t_total

    emb_pad = jnp.pad(embed, ((0, 0), (0, DP - embed.shape[1])))
    Wt, Ut, bt, Wdt, bdt = _prep_weights(W, U, b, Wd, bd)
    idx = _build_indices(input_1, input_2)

    n_phases = 2
    t_span = t_total // n_phases
    rows_span = b2 * t_span
    idx_rows_span = rows_span // 128
    g = _make_sc_gather(rows_span, DP)
    es = [
        g(emb_pad, idx[p * idx_rows_span:(p + 1) * idx_rows_span])
        .reshape(t_span, b2, DP)
        for p in range(n_phases)
    ]

    h, c = _make_lstm_call(b2, t_span, 10, first=True, last=False)(
        es[0], Wt, Ut, bt, Wdt, bdt)
    for p in range(1, n_phases - 1):
        h, c = _make_lstm_call(b2, t_span, 10, first=False, last=False)(
            es[p], Wt, Ut, bt, Wdt, bdt, h, c)
    out_t = _make_lstm_call(b2, t_span, 10, first=False, last=True)(
        es[-1], Wt, Ut, bt, Wdt, bdt, h, c)
    return out_t[:3].T


# speedup vs baseline: 1.0490x; 1.0135x over previous
import functools

import jax
import jax.numpy as jnp
from jax import lax
from jax.experimental import pallas as pl
from jax.experimental.pallas import tpu as pltpu
from jax.experimental.pallas import tpu_sc as plsc

D = 50
DP = 64
H = 64


def _make_sc_gather(n_rows, d, n_workers=32, k=8):
    sg_rows = k * 128
    n_sg = n_rows // sg_rows
    assert n_sg * sg_rows == n_rows
    n_outer = (n_sg + n_workers - 1) // n_workers

    mesh = plsc.VectorSubcoreMesh(core_axis_name="c", subcore_axis_name="s")

    @functools.partial(
        pl.kernel,
        mesh=mesh,
        compiler_params=pltpu.CompilerParams(use_tc_tiling_on_sc=False),
        out_type=jax.ShapeDtypeStruct((n_rows, d), jnp.float32),
        scratch_types=[
            pltpu.VMEM((k, 128), jnp.int32),
            pltpu.VMEM((sg_rows, d), jnp.float32),
            pltpu.SemaphoreType.DMA,
        ],
    )
    def gather(emb_hbm, idx_hbm, out_hbm, idx_v, rows_v, sem):
        wid = lax.axis_index("s") * 2 + lax.axis_index("c")

        def body(j, carry):
            sg = j * n_workers + wid

            @pl.when(sg < n_sg)
            def _():
                ir0 = pl.multiple_of(sg * k, 8)
                pltpu.sync_copy(idx_hbm.at[pl.ds(ir0, k)], idx_v)
                cps = [
                    pltpu.async_copy(
                        emb_hbm.at[idx_v.at[j2]],
                        rows_v.at[pl.ds(j2 * 128, 128)],
                        sem,
                    )
                    for j2 in range(k)
                ]
                for cp in cps:
                    cp.wait()
                r0 = pl.multiple_of(sg * sg_rows, 8)
                pltpu.sync_copy(rows_v, out_hbm.at[pl.ds(r0, sg_rows)])

            return carry

        lax.fori_loop(0, n_outer, body, 0)

    return gather


def _make_lstm_call(b2, t_span, tc, first, last, interpret=False):
    nt = t_span // tc
    half = b2 // 2

    def body(*refs):
        if first:
            (e_ref, wt_ref, ut_ref, bt_ref, wdt_ref, bdt_ref), rest = refs[:6], refs[6:]
        else:
            (e_ref, wt_ref, ut_ref, bt_ref, wdt_ref, bdt_ref, h0_ref, c0_ref), rest = refs[:8], refs[8:]
        if last:
            (out_ref, h_ref, c_ref) = rest
        else:
            (ho_ref, co_ref, h_ref, c_ref) = rest
        t_idx = pl.program_id(0)

        @pl.when(t_idx == 0)
        def _():
            if first:
                h_ref[...] = jnp.zeros((H, b2), jnp.float32)
                c_ref[...] = jnp.zeros((H, b2), jnp.float32)
            else:
                h_ref[...] = h0_ref[...]
                c_ref[...] = c0_ref[...]

        wt = wt_ref[...]
        ut = ut_ref[...]
        bt = bt_ref[...]

        def step(tt, hc):
            h, c = hc
            x = e_ref[tt]
            z = lax.dot_general(wt, x, (((1,), (1,)), ((), ())),
                                preferred_element_type=jnp.float32)
            z = z + jnp.dot(ut, h, preferred_element_type=jnp.float32) + bt
            gi = 0.5 + 0.5 * jnp.tanh(0.5 * z[0 * H:1 * H])
            gf = 0.5 + 0.5 * jnp.tanh(0.5 * z[1 * H:2 * H])
            gg = jnp.tanh(z[2 * H:3 * H])
            go = 0.5 + 0.5 * jnp.tanh(0.5 * z[3 * H:4 * H])
            c = gf * c + gi * gg
            h = go * jnp.tanh(c)
            return (h, c)

        h, c = lax.fori_loop(0, tc, step, (h_ref[...], c_ref[...]))
        h_ref[...] = h
        c_ref[...] = c

        @pl.when(t_idx == nt - 1)
        def _():
            if last:
                merged = jnp.concatenate([h[:, :half], h[:, half:]], axis=0)
                logits = (jnp.dot(wdt_ref[...], merged,
                                  preferred_element_type=jnp.float32)
                          + bdt_ref[...])
                m = jnp.max(logits, axis=0, keepdims=True)
                p = jnp.exp(logits - m)
                out_ref[...] = p / jnp.sum(p, axis=0, keepdims=True)
            else:
                ho_ref[...] = h
                co_ref[...] = c

    in_specs = [
        pl.BlockSpec((tc, b2, DP), lambda t: (t, 0, 0)),
        pl.BlockSpec((4 * H, DP), lambda t: (0, 0)),
        pl.BlockSpec((4 * H, H), lambda t: (0, 0)),
        pl.BlockSpec((4 * H, 1), lambda t: (0, 0)),
        pl.BlockSpec((8, 2 * H), lambda t: (0, 0)),
        pl.BlockSpec((8, 1), lambda t: (0, 0)),
    ]
    if not first:
        in_specs += [
            pl.BlockSpec((H, b2), lambda t: (0, 0)),
            pl.BlockSpec((H, b2), lambda t: (0, 0)),
        ]
    if last:
        out_specs = pl.BlockSpec((8, half), lambda t: (0, 0))
        out_shape = jax.ShapeDtypeStruct((8, half), jnp.float32)
    else:
        out_specs = [pl.BlockSpec((H, b2), lambda t: (0, 0))] * 2
        out_shape = [jax.ShapeDtypeStruct((H, b2), jnp.float32)] * 2

    return pl.pallas_call(
        body,
        grid=(nt,),
        in_specs=in_specs,
        out_specs=out_specs,
        out_shape=out_shape,
        scratch_shapes=[
            pltpu.VMEM((H, b2), jnp.float32),
            pltpu.VMEM((H, b2), jnp.float32),
        ],
        interpret=interpret,
    )


def _prep_weights(W, U, b, Wd, bd):
    Wt = jnp.pad(W, ((0, DP - W.shape[0]), (0, 0))).T
    Ut = U.T
    bt = b.reshape(-1, 1)
    Wdt = jnp.pad(Wd.T, ((0, 5), (0, 0)))
    bdt = jnp.concatenate([bd, jnp.full((5,), -1e30, bd.dtype)]).reshape(8, 1)
    return Wt, Ut, bt, Wdt, bdt


def _build_indices(input_1, input_2):
    return jnp.concatenate([input_1.T, input_2.T], axis=1).reshape(-1, 128).astype(jnp.int32)


def kernel(input_1, input_2, embed, W, U, b, Wd, bd):
    bsz, t_total = input_1.shape
    b2 = 2 * bsz
    n_rows = b2 * t_total

    emb_pad = jnp.pad(embed, ((0, 0), (0, DP - embed.shape[1])))
    Wt, Ut, bt, Wdt, bdt = _prep_weights(W, U, b, Wd, bd)
    idx = _build_indices(input_1, input_2)

    spans = [40, 60, 100]
    assert sum(spans) == t_total
    es, r0 = [], 0
    for ts in spans:
        nr = b2 * ts
        es.append(_make_sc_gather(nr, DP)(
            emb_pad, idx[r0 // 128:(r0 + nr) // 128]).reshape(ts, b2, DP))
        r0 += nr

    h, c = _make_lstm_call(b2, spans[0], 10, first=True, last=False)(
        es[0], Wt, Ut, bt, Wdt, bdt)
    for p in range(1, len(spans) - 1):
        h, c = _make_lstm_call(b2, spans[p], 10, first=False, last=False)(
            es[p], Wt, Ut, bt, Wdt, bdt, h, c)
    out_t = _make_lstm_call(b2, spans[-1], 10, first=False, last=True)(
        es[-1], Wt, Ut, bt, Wdt, bdt, h, c)
    return out_t[:3].T
